# baseline (speedup 1.0000x reference)
import jax
import jax.numpy as jnp
from jax.experimental import pallas as pl
from jax.experimental.pallas import tpu as pltpu

N = 10000
E = 160000
D = 256
G = 64
EPS = 1e-5


def _gcn_conv(x, edge_index, W, b):
    n = x.shape[0]
    h = x @ W
    loop = jnp.arange(n, dtype=edge_index.dtype)
    src = jnp.concatenate([edge_index[0], loop])
    dst = jnp.concatenate([edge_index[1], loop])
    deg = jnp.zeros((n,), dtype=h.dtype).at[dst].add(1.0)
    dis = jnp.where(deg > 0, jax.lax.rsqrt(deg), 0.0)
    coef = dis[src] * dis[dst]
    msg = h[src] * coef[:, None]
    out = jnp.zeros_like(h).at[dst].add(msg)
    return out + b


def _bn(x, gamma, beta):
    mu = jnp.mean(x, axis=0)
    var = jnp.var(x, axis=0)
    return gamma * (x - mu) / jnp.sqrt(var + EPS) + beta


def _mlp_kernel(p_ref, Wp1_ref, bp1_ref, Wp2_ref, bp2_ref, out_ref):
    p = p_ref[...]
    h = jnp.maximum(
        jnp.dot(p, Wp1_ref[...], preferred_element_type=jnp.float32)
        + bp1_ref[...], 0.0)
    out_ref[...] = (
        jnp.dot(h, Wp2_ref[...], preferred_element_type=jnp.float32)
        + bp2_ref[...])


def kernel(x, edge_index, batch, W1, b1, g1, be1, W2, b2, g2, be2,
           W3, b3, g3, be3, Wp1, bp1, Wp2, bp2):
    h = x
    for W, b, g, be in ((W1, b1, g1, be1), (W2, b2, g2, be2), (W3, b3, g3, be3)):
        h = _gcn_conv(h, edge_index, W, b)
        h = _bn(h, g, be)
        h = jax.nn.relu(h)
    p = jax.ops.segment_sum(h, batch, num_segments=G)
    return pl.pallas_call(
        _mlp_kernel,
        out_shape=jax.ShapeDtypeStruct((G, D), jnp.float32),
    )(p, Wp1, bp1.reshape(1, D), Wp2, bp2.reshape(1, D))


# speedup vs baseline: 11.5659x; 11.5659x over previous
import functools

import jax
import jax.numpy as jnp
from jax import lax
from jax.experimental import pallas as pl
from jax.experimental.pallas import tpu as pltpu
from jax.experimental.pallas import tpu_sc as plsc

N = 10000
E = 160000
D = 256
H = 128
G = 64
EPS = 1e-5

NC = 2
NS = 16

DEG_CH, DEG_CW = 40, 125
AGG_CH, AGG_CW = 80, 125

NPAD = 10240
ROWS_PER_TILE_DEG = NPAD // NS
ROWS_PER_TILE_AGG = NPAD // NS

_SC_MESH = plsc.VectorSubcoreMesh(
    core_axis_name="c", subcore_axis_name="s", num_cores=NC, num_subcores=NS)


@functools.partial(
    pl.kernel,
    out_type=jax.ShapeDtypeStruct((NC, NPAD, H), jnp.float32),
    mesh=_SC_MESH,
    scratch_types=[
        pltpu.VMEM((DEG_CH, DEG_CW), jnp.int32),
        pltpu.VMEM((DEG_CW, H), jnp.float32),
        pltpu.VMEM_SHARED((NPAD, H), jnp.float32),
    ],
)
def _sc_deg(dst_hbm, ones_hbm, zer_hbm, out_hbm, dst_v, ones_v, acc):
    cid = lax.axis_index("c")
    sid = lax.axis_index("s")
    wid = cid * NS + sid
    pltpu.sync_copy(dst_hbm.at[wid], dst_v)
    pltpu.sync_copy(ones_hbm, ones_v)
    pltpu.sync_copy(zer_hbm, acc.at[pl.ds(sid * ROWS_PER_TILE_DEG,
                                          ROWS_PER_TILE_DEG)])
    plsc.subcore_barrier()
    for j in range(DEG_CH):
        pltpu.sync_copy(ones_v, acc.at[dst_v.at[j]], add=True)
    plsc.subcore_barrier()
    pltpu.sync_copy(
        acc.at[pl.ds(sid * ROWS_PER_TILE_DEG, ROWS_PER_TILE_DEG)],
        out_hbm.at[cid, pl.ds(sid * ROWS_PER_TILE_DEG, ROWS_PER_TILE_DEG)])


@functools.partial(
    pl.kernel,
    out_type=jax.ShapeDtypeStruct((NC * NPAD, H), jnp.float32),
    mesh=_SC_MESH,
    scratch_types=[
        pltpu.VMEM((AGG_CH, AGG_CW), jnp.int32),
        pltpu.VMEM((AGG_CH, AGG_CW), jnp.int32),
        pltpu.VMEM((AGG_CW, H), jnp.float32),
        pltpu.VMEM_SHARED((NPAD, H), jnp.float32),
        pltpu.SemaphoreType.DMA,
    ],
)
def _sc_agg(yt_hbm, srcoff_hbm, dst_hbm, zrows_hbm, out_hbm,
            src_v, dst_v, rows, acc, sem):
    cid = lax.axis_index("c")
    sid = lax.axis_index("s")
    pltpu.sync_copy(srcoff_hbm.at[cid * NS + sid], src_v)
    pltpu.sync_copy(dst_hbm.at[sid], dst_v)
    pltpu.sync_copy(zrows_hbm, acc.at[pl.ds(sid * ROWS_PER_TILE_AGG,
                                            ROWS_PER_TILE_AGG)])
    plsc.subcore_barrier()
    for j in range(AGG_CH):
        pltpu.async_copy(yt_hbm.at[src_v.at[j]], rows, sem).wait()
        pltpu.sync_copy(rows, acc.at[dst_v.at[j]], add=True)
    plsc.subcore_barrier()
    pltpu.sync_copy(
        acc.at[pl.ds(sid * ROWS_PER_TILE_AGG, ROWS_PER_TILE_AGG)],
        out_hbm.at[pl.ds(cid * NPAD + sid * ROWS_PER_TILE_AGG,
                         ROWS_PER_TILE_AGG)])


def _tc_prep_body(x_ref, w_ref, dega_ref, yt_ref, dis_ref):
    deg = (dega_ref[0, :, 0:1] + dega_ref[1, :, 0:1] + 1.0)
    disf = lax.rsqrt(deg)
    dis_ref[...] = disf
    dis = disf[0:N, :]
    y = jnp.dot(x_ref[...], w_ref[...], preferred_element_type=jnp.float32)
    yt = y * dis
    yt_ref[0:N, :] = yt[:, 0:H]
    yt_ref[N:2 * N, :] = yt[:, H:D]


def _bn_relu_halves(agg_ref, yt_ref, dis, b_ref, g_ref, be_ref):
    hs = []
    for c in range(2):
        s = (agg_ref[c * NPAD:c * NPAD + N, :]
             + yt_ref[c * N:(c + 1) * N, :])
        z = dis * s + b_ref[:, c * H:(c + 1) * H]
        mu = jnp.mean(z, axis=0, keepdims=True)
        var = jnp.mean((z - mu) ** 2, axis=0, keepdims=True)
        zn = (g_ref[:, c * H:(c + 1) * H] * lax.rsqrt(var + EPS) * (z - mu)
              + be_ref[:, c * H:(c + 1) * H])
        hs.append(jnp.maximum(zn, 0.0))
    return hs


def _tc_layer_body(agg_ref, yt_ref, dis_ref, b_ref, g_ref, be_ref, w_ref,
                   out_ref):
    dis = dis_ref[0:N, :]
    hl, hr = _bn_relu_halves(agg_ref, yt_ref, dis, b_ref, g_ref, be_ref)
    y = (jnp.dot(hl, w_ref[0:H, :], preferred_element_type=jnp.float32)
         + jnp.dot(hr, w_ref[H:D, :], preferred_element_type=jnp.float32))
    yt = y * dis
    out_ref[0:N, :] = yt[:, 0:H]
    out_ref[N:2 * N, :] = yt[:, H:D]


def _tc_final_body(agg_ref, yt_ref, dis_ref, b_ref, g_ref, be_ref,
                   batch_ref, wp1_ref, bp1_ref, wp2_ref, bp2_ref, out_ref):
    dis = dis_ref[0:N, :]
    hl, hr = _bn_relu_halves(agg_ref, yt_ref, dis, b_ref, g_ref, be_ref)
    seg = lax.broadcasted_iota(jnp.int32, (G, N), 0)
    m = (seg == batch_ref[...]).astype(jnp.float32)
    pliters = jnp.dot(m, hl, preferred_element_type=jnp.float32)
    priters = jnp.dot(m, hr, preferred_element_type=jnp.float32)
    q = jnp.maximum(
        jnp.dot(pliters, wp1_ref[0:H, :], preferred_element_type=jnp.float32)
        + jnp.dot(priters, wp1_ref[H:D, :], preferred_element_type=jnp.float32)
        + bp1_ref[...], 0.0)
    out_ref[...] = (jnp.dot(q, wp2_ref[...], preferred_element_type=jnp.float32)
                    + bp2_ref[...])


_tc_prep = pl.pallas_call(
    _tc_prep_body,
    out_shape=(jax.ShapeDtypeStruct((NC * N, H), jnp.float32),
               jax.ShapeDtypeStruct((10240, 1), jnp.float32)),
)

_tc_layer = pl.pallas_call(
    _tc_layer_body,
    out_shape=jax.ShapeDtypeStruct((NC * N, H), jnp.float32),
)

_tc_final = pl.pallas_call(
    _tc_final_body,
    out_shape=jax.ShapeDtypeStruct((G, D), jnp.float32),
)


def kernel(x, edge_index, batch, W1, b1, g1, be1, W2, b2, g2, be2,
           W3, b3, g3, be3, Wp1, bp1, Wp2, bp2):
    src = edge_index[0]
    dst = edge_index[1]
    srcoff = jnp.concatenate([src, src + N]).reshape(NC * NS, AGG_CH, AGG_CW)
    dst_deg = dst.reshape(NC * NS, DEG_CH, DEG_CW)
    dst_agg = dst.reshape(NS, AGG_CH, AGG_CW)
    ones_r = jnp.ones((DEG_CW, H), jnp.float32)
    zrows = jnp.zeros((ROWS_PER_TILE_AGG, H), jnp.float32)
    b2d = [v.reshape(1, D) for v in (b1, g1, be1, b2, g2, be2, b3, g3, be3,
                                     bp1, bp2)]
    (b1r, g1r, be1r, b2r, g2r, be2r, b3r, g3r, be3r, bp1r, bp2r) = b2d
    batch2 = batch.reshape(1, N)

    dega = _sc_deg(dst_deg, ones_r, zrows)
    yt1, dis = _tc_prep(x, W1, dega)
    agg1 = _sc_agg(yt1, srcoff, dst_agg, zrows)
    yt2 = _tc_layer(agg1, yt1, dis, b1r, g1r, be1r, W2)
    agg2 = _sc_agg(yt2, srcoff, dst_agg, zrows)
    yt3 = _tc_layer(agg2, yt2, dis, b2r, g2r, be2r, W3)
    agg3 = _sc_agg(yt3, srcoff, dst_agg, zrows)
    return _tc_final(agg3, yt3, dis, b3r, g3r, be3r, batch2,
                     Wp1, bp1r, Wp2, bp2r)


# speedup vs baseline: 15.3781x; 1.3296x over previous
import functools

import jax
import jax.numpy as jnp
from jax import lax
from jax.experimental import pallas as pl
from jax.experimental.pallas import tpu as pltpu
from jax.experimental.pallas import tpu_sc as plsc

N = 10000
E = 160000
D = 256
H = 128
G = 64
EPS = 1e-5

NC = 2
NS = 16

DEG_CH, DEG_CW = 40, 125
AGG_CH, AGG_CW = 80, 125
AGG_PGC = 8
AGG_NP = AGG_CH // AGG_PGC

NPAD = 10240
ROWS_PER_TILE_DEG = NPAD // NS
ROWS_PER_TILE_AGG = NPAD // NS

_SC_MESH = plsc.VectorSubcoreMesh(
    core_axis_name="c", subcore_axis_name="s", num_cores=NC, num_subcores=NS)


@functools.partial(
    pl.kernel,
    out_type=jax.ShapeDtypeStruct((NC, NPAD, H), jnp.float32),
    mesh=_SC_MESH,
    scratch_types=[
        pltpu.VMEM((DEG_CH, DEG_CW), jnp.int32),
        pltpu.VMEM((DEG_CW, H), jnp.float32),
        pltpu.VMEM_SHARED((NPAD, H), jnp.float32),
    ],
)
def _sc_deg(dst_hbm, ones_hbm, zer_hbm, out_hbm, dst_v, ones_v, acc):
    cid = lax.axis_index("c")
    sid = lax.axis_index("s")
    wid = cid * NS + sid
    pltpu.sync_copy(dst_hbm.at[wid], dst_v)
    pltpu.sync_copy(ones_hbm, ones_v)
    for z in range(8):
        pltpu.sync_copy(
            zer_hbm,
            acc.at[pl.ds(sid * ROWS_PER_TILE_DEG + z * (ROWS_PER_TILE_DEG // 8),
                         ROWS_PER_TILE_DEG // 8)])
    plsc.subcore_barrier()
    for j in range(DEG_CH):
        pltpu.sync_copy(ones_v, acc.at[dst_v.at[j]], add=True)
    plsc.subcore_barrier()
    pltpu.sync_copy(
        acc.at[pl.ds(sid * ROWS_PER_TILE_DEG, ROWS_PER_TILE_DEG)],
        out_hbm.at[cid, pl.ds(sid * ROWS_PER_TILE_DEG, ROWS_PER_TILE_DEG)])


@functools.partial(
    pl.kernel,
    out_type=jax.ShapeDtypeStruct((NC * NPAD, H), jnp.float32),
    mesh=_SC_MESH,
    scratch_types=[
        pltpu.VMEM((2, AGG_PGC, AGG_CW), jnp.int32),
        pltpu.VMEM((2, AGG_PGC, AGG_CW), jnp.int32),
        pltpu.VMEM((AGG_CW, H), jnp.float32),
        pltpu.VMEM((AGG_CW, H), jnp.float32),
        pltpu.VMEM_SHARED((NPAD, H), jnp.float32),
        pltpu.SemaphoreType.DMA,
        pltpu.SemaphoreType.DMA,
        pltpu.SemaphoreType.DMA,
    ],
)
def _sc_agg(yt_hbm, srcoff_hbm, dst_hbm, zrows_hbm, out_hbm,
            srcpg, dstpg, rows0, rows1, acc, gsem, ssem, psem):
    cid = lax.axis_index("c")
    sid = lax.axis_index("s")
    w = cid * NS + sid

    def prefetch(p):
        sl = pl.ds(p * AGG_PGC, AGG_PGC)
        return (pltpu.async_copy(srcoff_hbm.at[w, sl], srcpg.at[p % 2], psem),
                pltpu.async_copy(dst_hbm.at[sid, sl], dstpg.at[p % 2], psem))

    pf = {0: prefetch(0), 1: prefetch(1)}
    for z in range(8):
        pltpu.sync_copy(
            zrows_hbm,
            acc.at[pl.ds(sid * ROWS_PER_TILE_AGG + z * (ROWS_PER_TILE_AGG // 8),
                         ROWS_PER_TILE_AGG // 8)])
    plsc.subcore_barrier()
    bufs = (rows0, rows1)
    gats = [None] * AGG_CH
    scas = [None] * AGG_CH
    for j in range(AGG_CH):
        p, k = divmod(j, AGG_PGC)
        if k == 0 and p in pf:
            for d in pf.pop(p):
                d.wait()
        if j >= 2:
            scas[j - 2].wait()
        gats[j] = pltpu.async_copy(yt_hbm.at[srcpg.at[p % 2, k]],
                                   bufs[j % 2], gsem)
        if j >= 1:
            gats[j - 1].wait()
            pm1, km1 = divmod(j - 1, AGG_PGC)
            scas[j - 1] = pltpu.async_copy(
                bufs[(j - 1) % 2], acc.at[dstpg.at[pm1 % 2, km1]], ssem,
                add=True)
        if k == 1 and p + 1 < AGG_NP and p + 1 not in pf:
            pf[p + 1] = prefetch(p + 1)
    gats[AGG_CH - 1].wait()
    scas[AGG_CH - 1] = pltpu.async_copy(
        bufs[(AGG_CH - 1) % 2],
        acc.at[dstpg.at[(AGG_NP - 1) % 2, AGG_PGC - 1]], ssem, add=True)
    scas[AGG_CH - 2].wait()
    scas[AGG_CH - 1].wait()
    plsc.subcore_barrier()
    pltpu.sync_copy(
        acc.at[pl.ds(sid * ROWS_PER_TILE_AGG, ROWS_PER_TILE_AGG)],
        out_hbm.at[pl.ds(cid * NPAD + sid * ROWS_PER_TILE_AGG,
                         ROWS_PER_TILE_AGG)])


def _tc_prep_body(x_ref, w_ref, dega_ref, yt_ref, dis_ref):
    deg = (dega_ref[0, :, 0:1] + dega_ref[1, :, 0:1] + 1.0)
    disf = lax.rsqrt(deg)
    dis_ref[...] = disf
    dis = disf[0:N, :]
    y = jnp.dot(x_ref[...], w_ref[...], preferred_element_type=jnp.float32)
    yt = y * dis
    yt_ref[0:N, :] = yt[:, 0:H]
    yt_ref[N:2 * N, :] = yt[:, H:D]


def _bn_relu_halves(agg_ref, yt_ref, dis, b_ref, g_ref, be_ref):
    hs = []
    for c in range(2):
        s = (agg_ref[c * NPAD:c * NPAD + N, :]
             + yt_ref[c * N:(c + 1) * N, :])
        z = dis * s + b_ref[:, c * H:(c + 1) * H]
        mu = jnp.mean(z, axis=0, keepdims=True)
        var = jnp.mean((z - mu) ** 2, axis=0, keepdims=True)
        zn = (g_ref[:, c * H:(c + 1) * H] * lax.rsqrt(var + EPS) * (z - mu)
              + be_ref[:, c * H:(c + 1) * H])
        hs.append(jnp.maximum(zn, 0.0))
    return hs


def _tc_layer_body(agg_ref, yt_ref, dis_ref, b_ref, g_ref, be_ref, w_ref,
                   out_ref):
    dis = dis_ref[0:N, :]
    hl, hr = _bn_relu_halves(agg_ref, yt_ref, dis, b_ref, g_ref, be_ref)
    y = (jnp.dot(hl, w_ref[0:H, :], preferred_element_type=jnp.float32)
         + jnp.dot(hr, w_ref[H:D, :], preferred_element_type=jnp.float32))
    yt = y * dis
    out_ref[0:N, :] = yt[:, 0:H]
    out_ref[N:2 * N, :] = yt[:, H:D]


def _tc_final_body(agg_ref, yt_ref, dis_ref, b_ref, g_ref, be_ref,
                   batch_ref, wp1_ref, bp1_ref, wp2_ref, bp2_ref, out_ref):
    dis = dis_ref[0:N, :]
    hl, hr = _bn_relu_halves(agg_ref, yt_ref, dis, b_ref, g_ref, be_ref)
    seg = lax.broadcasted_iota(jnp.int32, (G, N), 0)
    m = (seg == batch_ref[...]).astype(jnp.float32)
    pliters = jnp.dot(m, hl, preferred_element_type=jnp.float32)
    priters = jnp.dot(m, hr, preferred_element_type=jnp.float32)
    q = jnp.maximum(
        jnp.dot(pliters, wp1_ref[0:H, :], preferred_element_type=jnp.float32)
        + jnp.dot(priters, wp1_ref[H:D, :], preferred_element_type=jnp.float32)
        + bp1_ref[...], 0.0)
    out_ref[...] = (jnp.dot(q, wp2_ref[...], preferred_element_type=jnp.float32)
                    + bp2_ref[...])


_tc_prep = pl.pallas_call(
    _tc_prep_body,
    out_shape=(jax.ShapeDtypeStruct((NC * N, H), jnp.float32),
               jax.ShapeDtypeStruct((10240, 1), jnp.float32)),
)

_tc_layer = pl.pallas_call(
    _tc_layer_body,
    out_shape=jax.ShapeDtypeStruct((NC * N, H), jnp.float32),
)

_tc_final = pl.pallas_call(
    _tc_final_body,
    out_shape=jax.ShapeDtypeStruct((G, D), jnp.float32),
)


def kernel(x, edge_index, batch, W1, b1, g1, be1, W2, b2, g2, be2,
           W3, b3, g3, be3, Wp1, bp1, Wp2, bp2):
    src = edge_index[0]
    dst = edge_index[1]
    srcoff = jnp.concatenate([src, src + N]).reshape(NC * NS, AGG_CH, AGG_CW)
    dst_deg = dst.reshape(NC * NS, DEG_CH, DEG_CW)
    dst_agg = dst.reshape(NS, AGG_CH, AGG_CW)
    ones_r = jnp.ones((DEG_CW, H), jnp.float32)
    zrows = jnp.zeros((ROWS_PER_TILE_AGG // 8, H), jnp.float32)
    b2d = [v.reshape(1, D) for v in (b1, g1, be1, b2, g2, be2, b3, g3, be3,
                                     bp1, bp2)]
    (b1r, g1r, be1r, b2r, g2r, be2r, b3r, g3r, be3r, bp1r, bp2r) = b2d
    batch2 = batch.reshape(1, N)

    dega = _sc_deg(dst_deg, ones_r, zrows)
    yt1, dis = _tc_prep(x, W1, dega)
    agg1 = _sc_agg(yt1, srcoff, dst_agg, zrows)
    yt2 = _tc_layer(agg1, yt1, dis, b1r, g1r, be1r, W2)
    agg2 = _sc_agg(yt2, srcoff, dst_agg, zrows)
    yt3 = _tc_layer(agg2, yt2, dis, b2r, g2r, be2r, W3)
    agg3 = _sc_agg(yt3, srcoff, dst_agg, zrows)
    return _tc_final(agg3, yt3, dis, b3r, g3r, be3r, batch2,
                     Wp1, bp1r, Wp2, bp2r)


# speedup vs baseline: 16.7395x; 1.0885x over previous
import functools

import jax
import jax.numpy as jnp
from jax import lax
from jax.experimental import pallas as pl
from jax.experimental.pallas import tpu as pltpu
from jax.experimental.pallas import tpu_sc as plsc

N = 10000
E = 160000
D = 256
H = 128
G = 64
EPS = 1e-5

NC = 2
NS = 16

DEG_CH, DEG_CW = 40, 125
AGG_CH, AGG_CW = 125, 80
AGG_PGC = 5
AGG_NP = AGG_CH // AGG_PGC
AGG_NB = 4

NPAD = 10240
ROWS_PER_TILE_DEG = NPAD // NS
ROWS_PER_TILE_AGG = NPAD // NS

_SC_MESH = plsc.VectorSubcoreMesh(
    core_axis_name="c", subcore_axis_name="s", num_cores=NC, num_subcores=NS)


DEG_W = 128


@functools.partial(
    pl.kernel,
    out_type=jax.ShapeDtypeStruct((NC, NPAD, DEG_W), jnp.float32),
    mesh=_SC_MESH,
    scratch_types=[
        pltpu.VMEM((DEG_CH, DEG_CW), jnp.int32),
        pltpu.VMEM((DEG_CW, DEG_W), jnp.float32),
        pltpu.VMEM_SHARED((NPAD, DEG_W), jnp.float32),
    ],
)
def _sc_deg(dst_hbm, ones_hbm, zer_hbm, out_hbm, dst_v, ones_v, acc):
    cid = lax.axis_index("c")
    sid = lax.axis_index("s")
    wid = cid * NS + sid
    pltpu.sync_copy(dst_hbm.at[wid], dst_v)
    pltpu.sync_copy(ones_hbm, ones_v)
    for z in range(8):
        pltpu.sync_copy(
            zer_hbm,
            acc.at[pl.ds(sid * ROWS_PER_TILE_DEG + z * (ROWS_PER_TILE_DEG // 8),
                         ROWS_PER_TILE_DEG // 8)])
    plsc.subcore_barrier()
    for j in range(DEG_CH):
        pltpu.sync_copy(ones_v, acc.at[dst_v.at[j]], add=True)
    plsc.subcore_barrier()
    pltpu.sync_copy(
        acc.at[pl.ds(sid * ROWS_PER_TILE_DEG, ROWS_PER_TILE_DEG)],
        out_hbm.at[cid, pl.ds(sid * ROWS_PER_TILE_DEG, ROWS_PER_TILE_DEG)])


@functools.partial(
    pl.kernel,
    out_type=jax.ShapeDtypeStruct((NC * NPAD, H), jnp.float32),
    mesh=_SC_MESH,
    scratch_types=[
        pltpu.VMEM((3, AGG_PGC, AGG_CW), jnp.int32),
        pltpu.VMEM((3, AGG_PGC, AGG_CW), jnp.int32),
        pltpu.VMEM((AGG_NB, AGG_CW, H), jnp.float32),
        pltpu.VMEM_SHARED((NPAD, H), jnp.float32),
        pltpu.SemaphoreType.DMA,
        pltpu.SemaphoreType.DMA,
        pltpu.SemaphoreType.DMA,
    ],
)
def _sc_agg(yt_hbm, srcoff_hbm, dst_hbm, zrows_hbm, out_hbm,
            srcpg, dstpg, rowsb, acc, gsem, ssem, psem):
    cid = lax.axis_index("c")
    sid = lax.axis_index("s")
    w = cid * NS + sid

    def prefetch(q):
        return (pltpu.async_copy(srcoff_hbm.at[w, q], srcpg.at[q % 3], psem),
                pltpu.async_copy(dst_hbm.at[sid, q], dstpg.at[q % 3], psem))

    def gather(c):
        q, r = divmod(c, AGG_PGC)
        return pltpu.async_copy(yt_hbm.at[srcpg.at[q % 3, r]],
                                rowsb.at[c % AGG_NB], gsem)

    def scatter(c):
        q, r = divmod(c, AGG_PGC)
        return pltpu.async_copy(rowsb.at[c % AGG_NB],
                                acc.at[dstpg.at[q % 3, r]], ssem, add=True)

    pf = {q: prefetch(q) for q in (0, 1, 2)}
    for z in range(8):
        pltpu.sync_copy(
            zrows_hbm,
            acc.at[pl.ds(sid * ROWS_PER_TILE_AGG + z * (ROWS_PER_TILE_AGG // 8),
                         ROWS_PER_TILE_AGG // 8)])
    plsc.subcore_barrier()
    gats = [None] * AGG_CH
    scas = [None] * AGG_CH
    for d in pf.pop(0):
        d.wait()
    gats[0] = gather(0)
    gats[1] = gather(1)
    for j in range(AGG_CH):
        if j >= 2:
            scas[j - 2].wait()
        c = j + 2
        if c < AGG_CH:
            q, r = divmod(c, AGG_PGC)
            if r == 0 and q in pf:
                for d in pf.pop(q):
                    d.wait()
            gats[c] = gather(c)
            if r == 3 and q + 2 < AGG_NP and (q + 2) not in pf:
                pf[q + 2] = prefetch(q + 2)
        gats[j].wait()
        scas[j] = scatter(j)
    scas[AGG_CH - 2].wait()
    scas[AGG_CH - 1].wait()
    plsc.subcore_barrier()
    pltpu.sync_copy(
        acc.at[pl.ds(sid * ROWS_PER_TILE_AGG, ROWS_PER_TILE_AGG)],
        out_hbm.at[pl.ds(cid * NPAD + sid * ROWS_PER_TILE_AGG,
                         ROWS_PER_TILE_AGG)])


def _tc_prep_body(x_ref, w_ref, dega_ref, yt_ref, dis_ref):
    deg = (dega_ref[0, :, 0:1] + dega_ref[1, :, 0:1] + 1.0)
    disf = lax.rsqrt(deg)
    dis_ref[...] = disf
    dis = disf[0:N, :]
    y = jnp.dot(x_ref[...], w_ref[...], preferred_element_type=jnp.float32)
    yt = y * dis
    yt_ref[0:N, :] = yt[:, 0:H]
    yt_ref[N:2 * N, :] = yt[:, H:D]


def _bn_relu_halves(agg_ref, yt_ref, dis, b_ref, g_ref, be_ref):
    hs = []
    for c in range(2):
        s = (agg_ref[c * NPAD:c * NPAD + N, :]
             + yt_ref[c * N:(c + 1) * N, :])
        z = dis * s + b_ref[:, c * H:(c + 1) * H]
        mu = jnp.mean(z, axis=0, keepdims=True)
        var = jnp.mean((z - mu) ** 2, axis=0, keepdims=True)
        zn = (g_ref[:, c * H:(c + 1) * H] * lax.rsqrt(var + EPS) * (z - mu)
              + be_ref[:, c * H:(c + 1) * H])
        hs.append(jnp.maximum(zn, 0.0))
    return hs


def _tc_layer_body(agg_ref, yt_ref, dis_ref, b_ref, g_ref, be_ref, w_ref,
                   out_ref):
    dis = dis_ref[0:N, :]
    hl, hr = _bn_relu_halves(agg_ref, yt_ref, dis, b_ref, g_ref, be_ref)
    y = (jnp.dot(hl, w_ref[0:H, :], preferred_element_type=jnp.float32)
         + jnp.dot(hr, w_ref[H:D, :], preferred_element_type=jnp.float32))
    yt = y * dis
    out_ref[0:N, :] = yt[:, 0:H]
    out_ref[N:2 * N, :] = yt[:, H:D]


def _tc_final_body(agg_ref, yt_ref, dis_ref, b_ref, g_ref, be_ref,
                   batch_ref, wp1_ref, bp1_ref, wp2_ref, bp2_ref, out_ref):
    dis = dis_ref[0:N, :]
    hl, hr = _bn_relu_halves(agg_ref, yt_ref, dis, b_ref, g_ref, be_ref)
    seg = lax.broadcasted_iota(jnp.int32, (G, N), 0)
    m = (seg == batch_ref[...]).astype(jnp.float32)
    pliters = jnp.dot(m, hl, preferred_element_type=jnp.float32)
    priters = jnp.dot(m, hr, preferred_element_type=jnp.float32)
    q = jnp.maximum(
        jnp.dot(pliters, wp1_ref[0:H, :], preferred_element_type=jnp.float32)
        + jnp.dot(priters, wp1_ref[H:D, :], preferred_element_type=jnp.float32)
        + bp1_ref[...], 0.0)
    out_ref[...] = (jnp.dot(q, wp2_ref[...], preferred_element_type=jnp.float32)
                    + bp2_ref[...])


_tc_prep = pl.pallas_call(
    _tc_prep_body,
    out_shape=(jax.ShapeDtypeStruct((NC * N, H), jnp.float32),
               jax.ShapeDtypeStruct((10240, 1), jnp.float32)),
)

_tc_layer = pl.pallas_call(
    _tc_layer_body,
    out_shape=jax.ShapeDtypeStruct((NC * N, H), jnp.float32),
)

_tc_final = pl.pallas_call(
    _tc_final_body,
    out_shape=jax.ShapeDtypeStruct((G, D), jnp.float32),
)


def kernel(x, edge_index, batch, W1, b1, g1, be1, W2, b2, g2, be2,
           W3, b3, g3, be3, Wp1, bp1, Wp2, bp2):
    src = edge_index[0]
    dst = edge_index[1]
    srcoff = jnp.concatenate([src, src + N]).reshape(
        NC * NS, AGG_NP, AGG_PGC, AGG_CW)
    dst_deg = dst.reshape(NC * NS, DEG_CH, DEG_CW)
    dst_agg = dst.reshape(NS, AGG_NP, AGG_PGC, AGG_CW)
    ones_r = jnp.ones((DEG_CW, DEG_W), jnp.float32)
    zer_d = jnp.zeros((ROWS_PER_TILE_DEG // 8, DEG_W), jnp.float32)
    zrows = jnp.zeros((ROWS_PER_TILE_AGG // 8, H), jnp.float32)
    b2d = [v.reshape(1, D) for v in (b1, g1, be1, b2, g2, be2, b3, g3, be3,
                                     bp1, bp2)]
    (b1r, g1r, be1r, b2r, g2r, be2r, b3r, g3r, be3r, bp1r, bp2r) = b2d
    batch2 = batch.reshape(1, N)

    dega = _sc_deg(dst_deg, ones_r, zer_d)
    yt1, dis = _tc_prep(x, W1, dega)
    agg1 = _sc_agg(yt1, srcoff, dst_agg, zrows)
    yt2 = _tc_layer(agg1, yt1, dis, b1r, g1r, be1r, W2)
    agg2 = _sc_agg(yt2, srcoff, dst_agg, zrows)
    yt3 = _tc_layer(agg2, yt2, dis, b2r, g2r, be2r, W3)
    agg3 = _sc_agg(yt3, srcoff, dst_agg, zrows)
    return _tc_final(agg3, yt3, dis, b3r, g3r, be3r, batch2,
                     Wp1, bp1r, Wp2, bp2r)


# speedup vs baseline: 18.9334x; 1.1311x over previous
import functools

import jax
import jax.numpy as jnp
from jax import lax
from jax.experimental import pallas as pl
from jax.experimental.pallas import tpu as pltpu
from jax.experimental.pallas import tpu_sc as plsc

N = 10000
E = 160000
D = 256
H = 128
G = 64
EPS = 1e-5

NC = 2
NS = 16

DEG_CH, DEG_CW = 40, 125
AGG_CH, AGG_CW = 125, 80
AGG_PGC = 5
AGG_NP = AGG_CH // AGG_PGC
AGG_NB = 4

NPAD = 10240
ROWS_PER_TILE_DEG = NPAD // NS
ROWS_PER_TILE_AGG = NPAD // NS

_SC_MESH = plsc.VectorSubcoreMesh(
    core_axis_name="c", subcore_axis_name="s", num_cores=NC, num_subcores=NS)


@functools.partial(
    pl.kernel,
    out_type=jax.ShapeDtypeStruct((NC, NPAD, H), jnp.float32),
    mesh=_SC_MESH,
    scratch_types=[
        pltpu.VMEM((DEG_CH, DEG_CW), jnp.int32),
        pltpu.VMEM((DEG_CW, H), jnp.float32),
        pltpu.VMEM_SHARED((NPAD, H), jnp.float32),
    ],
)
def _sc_deg(dst_hbm, ones_hbm, zer_hbm, out_hbm, dst_v, ones_v, acc):
    cid = lax.axis_index("c")
    sid = lax.axis_index("s")
    wid = cid * NS + sid
    pltpu.sync_copy(dst_hbm.at[wid], dst_v)
    pltpu.sync_copy(zer_hbm, ones_v.at[pl.ds(0, ROWS_PER_TILE_DEG // 8)])
    for z in range(8):
        pltpu.sync_copy(
            ones_v.at[pl.ds(0, ROWS_PER_TILE_DEG // 8)],
            acc.at[pl.ds(sid * ROWS_PER_TILE_DEG + z * (ROWS_PER_TILE_DEG // 8),
                         ROWS_PER_TILE_DEG // 8)])
    pltpu.sync_copy(ones_hbm, ones_v)
    plsc.subcore_barrier()
    for j in range(DEG_CH):
        pltpu.sync_copy(ones_v, acc.at[dst_v.at[j]], add=True)
    plsc.subcore_barrier()
    pltpu.sync_copy(
        acc.at[pl.ds(sid * ROWS_PER_TILE_DEG, ROWS_PER_TILE_DEG)],
        out_hbm.at[cid, pl.ds(sid * ROWS_PER_TILE_DEG, ROWS_PER_TILE_DEG)])


@functools.partial(
    pl.kernel,
    out_type=jax.ShapeDtypeStruct((NC * NPAD, H), jnp.float32),
    mesh=_SC_MESH,
    scratch_types=[
        pltpu.VMEM((3, AGG_PGC, AGG_CW), jnp.int32),
        pltpu.VMEM((3, AGG_PGC, AGG_CW), jnp.int32),
        pltpu.VMEM((AGG_NB, AGG_CW, H), jnp.float32),
        pltpu.VMEM_SHARED((NPAD, H), jnp.float32),
        pltpu.SemaphoreType.DMA,
        pltpu.SemaphoreType.DMA,
        pltpu.SemaphoreType.DMA,
    ],
)
def _sc_agg(yt_hbm, srcoff_hbm, dst_hbm, zrows_hbm, out_hbm,
            srcpg, dstpg, rowsb, acc, gsem, ssem, psem):
    cid = lax.axis_index("c")
    sid = lax.axis_index("s")
    w = cid * NS + sid

    def prefetch(q):
        return (pltpu.async_copy(srcoff_hbm.at[w, q], srcpg.at[q % 3], psem),
                pltpu.async_copy(dst_hbm.at[sid, q], dstpg.at[q % 3], psem))

    def gather(c):
        q, r = divmod(c, AGG_PGC)
        return pltpu.async_copy(yt_hbm.at[srcpg.at[q % 3, r]],
                                rowsb.at[c % AGG_NB], gsem)

    def scatter(c):
        q, r = divmod(c, AGG_PGC)
        return pltpu.async_copy(rowsb.at[c % AGG_NB],
                                acc.at[dstpg.at[q % 3, r]], ssem, add=True)

    pf = {q: prefetch(q) for q in (0, 1, 2)}
    pltpu.sync_copy(zrows_hbm, rowsb.at[0])
    for z in range(8):
        pltpu.sync_copy(
            rowsb.at[0],
            acc.at[pl.ds(sid * ROWS_PER_TILE_AGG + z * (ROWS_PER_TILE_AGG // 8),
                         ROWS_PER_TILE_AGG // 8)])
    plsc.subcore_barrier()
    gats = [None] * AGG_CH
    scas = [None] * AGG_CH
    for d in pf.pop(0):
        d.wait()
    gats[0] = gather(0)
    gats[1] = gather(1)
    for j in range(AGG_CH):
        if j >= 2:
            scas[j - 2].wait()
        c = j + 2
        if c < AGG_CH:
            q, r = divmod(c, AGG_PGC)
            if r == 0 and q in pf:
                for d in pf.pop(q):
                    d.wait()
            gats[c] = gather(c)
            if r == 3 and q + 2 < AGG_NP and (q + 2) not in pf:
                pf[q + 2] = prefetch(q + 2)
        gats[j].wait()
        scas[j] = scatter(j)
    scas[AGG_CH - 2].wait()
    scas[AGG_CH - 1].wait()
    plsc.subcore_barrier()
    pltpu.sync_copy(
        acc.at[pl.ds(sid * ROWS_PER_TILE_AGG, ROWS_PER_TILE_AGG)],
        out_hbm.at[pl.ds(cid * NPAD + sid * ROWS_PER_TILE_AGG,
                         ROWS_PER_TILE_AGG)])


def _tc_prep_body(x_ref, w_ref, dega_ref, yt_ref, dis_ref):
    deg = (dega_ref[0, :, 0:1] + dega_ref[1, :, 0:1] + 1.0)
    disf = lax.rsqrt(deg)
    dis_ref[...] = disf
    dis = disf[0:N, :]
    y = jnp.dot(x_ref[...], w_ref[...], preferred_element_type=jnp.float32)
    yt = y * dis
    yt_ref[0:N, :] = yt[:, 0:H]
    yt_ref[N:2 * N, :] = yt[:, H:D]


def _bn_relu_halves(agg_ref, yt_ref, dis, b_ref, g_ref, be_ref):
    hs = []
    for c in range(2):
        s = (agg_ref[c * NPAD:c * NPAD + N, :]
             + yt_ref[c * N:(c + 1) * N, :])
        z = dis * s + b_ref[:, c * H:(c + 1) * H]
        mu = jnp.mean(z, axis=0, keepdims=True)
        var = jnp.mean((z - mu) ** 2, axis=0, keepdims=True)
        zn = (g_ref[:, c * H:(c + 1) * H] * lax.rsqrt(var + EPS) * (z - mu)
              + be_ref[:, c * H:(c + 1) * H])
        hs.append(jnp.maximum(zn, 0.0))
    return hs


def _tc_layer_body(agg_ref, yt_ref, dis_ref, b_ref, g_ref, be_ref, w_ref,
                   out_ref):
    dis = dis_ref[0:N, :]
    hl, hr = _bn_relu_halves(agg_ref, yt_ref, dis, b_ref, g_ref, be_ref)
    y = (jnp.dot(hl, w_ref[0:H, :], preferred_element_type=jnp.float32)
         + jnp.dot(hr, w_ref[H:D, :], preferred_element_type=jnp.float32))
    yt = y * dis
    out_ref[0:N, :] = yt[:, 0:H]
    out_ref[N:2 * N, :] = yt[:, H:D]


def _tc_final_body(agg_ref, yt_ref, dis_ref, b_ref, g_ref, be_ref,
                   batch_ref, wp1_ref, bp1_ref, wp2_ref, bp2_ref, out_ref):
    dis = dis_ref[0:N, :]
    hl, hr = _bn_relu_halves(agg_ref, yt_ref, dis, b_ref, g_ref, be_ref)
    seg = lax.broadcasted_iota(jnp.int32, (G, N), 0)
    m = (seg == batch_ref[...]).astype(jnp.float32)
    pliters = jnp.dot(m, hl, preferred_element_type=jnp.float32)
    priters = jnp.dot(m, hr, preferred_element_type=jnp.float32)
    q = jnp.maximum(
        jnp.dot(pliters, wp1_ref[0:H, :], preferred_element_type=jnp.float32)
        + jnp.dot(priters, wp1_ref[H:D, :], preferred_element_type=jnp.float32)
        + bp1_ref[...], 0.0)
    out_ref[...] = (jnp.dot(q, wp2_ref[...], preferred_element_type=jnp.float32)
                    + bp2_ref[...])


_tc_prep = pl.pallas_call(
    _tc_prep_body,
    out_shape=(jax.ShapeDtypeStruct((NC * N, H), jnp.float32),
               jax.ShapeDtypeStruct((10240, 1), jnp.float32)),
)

_tc_layer = pl.pallas_call(
    _tc_layer_body,
    out_shape=jax.ShapeDtypeStruct((NC * N, H), jnp.float32),
)

_tc_final = pl.pallas_call(
    _tc_final_body,
    out_shape=jax.ShapeDtypeStruct((G, D), jnp.float32),
)


def kernel(x, edge_index, batch, W1, b1, g1, be1, W2, b2, g2, be2,
           W3, b3, g3, be3, Wp1, bp1, Wp2, bp2):
    src = edge_index[0]
    dst = edge_index[1]
    srcoff = jnp.concatenate([src, src + N]).reshape(
        NC * NS, AGG_NP, AGG_PGC, AGG_CW)
    dst_deg = dst.reshape(NC * NS, DEG_CH, DEG_CW)
    dst_agg = dst.reshape(NS, AGG_NP, AGG_PGC, AGG_CW)
    ones_r = jnp.ones((DEG_CW, H), jnp.float32)
    zrows = jnp.zeros((ROWS_PER_TILE_AGG // 8, H), jnp.float32)
    b2d = [v.reshape(1, D) for v in (b1, g1, be1, b2, g2, be2, b3, g3, be3,
                                     bp1, bp2)]
    (b1r, g1r, be1r, b2r, g2r, be2r, b3r, g3r, be3r, bp1r, bp2r) = b2d
    batch2 = batch.reshape(1, N)

    dega = _sc_deg(dst_deg, ones_r, zrows)
    yt1, dis = _tc_prep(x, W1, dega)
    agg1 = _sc_agg(yt1, srcoff, dst_agg, zrows)
    yt2 = _tc_layer(agg1, yt1, dis, b1r, g1r, be1r, W2)
    agg2 = _sc_agg(yt2, srcoff, dst_agg, zrows)
    yt3 = _tc_layer(agg2, yt2, dis, b2r, g2r, be2r, W3)
    agg3 = _sc_agg(yt3, srcoff, dst_agg, zrows)
    return _tc_final(agg3, yt3, dis, b3r, g3r, be3r, batch2,
                     Wp1, bp1r, Wp2, bp2r)
